# baseline (speedup 1.0000x reference)
import functools

import jax
import jax.numpy as jnp
from jax import lax
from jax.experimental import pallas as pl
from jax.experimental.pallas import tpu as pltpu
from jax.experimental.pallas import tpu_sc as plsc

_THRESH = -3.0


def _pass1_body(dA_ref, dB_ref, mA_ref, mB_ref,
                scores_ref, rmax_ref, rarg_ref, cmax_ref, *, precision):
    dB = dB_ref[0]
    nB = jnp.sqrt(jnp.sum(dB * dB, axis=-1, keepdims=True))
    dBn = dB / (nB + 1e-12)

    dA = dA_ref[0]
    nA = jnp.sqrt(jnp.sum(dA * dA, axis=-1, keepdims=True))
    d = dA.shape[-1]
    dAn = (dA / (nA + 1e-12)) * (float(d) ** 0.5)

    s = lax.dot_general(dAn, dBn, (((1,), (1,)), ((), ())),
                        precision=precision,
                        preferred_element_type=jnp.float32)
    s = (s + mA_ref[0, 0][:, None]) + mB_ref[0, 0][None, :]
    scores_ref[0] = s

    m, n = s.shape
    rmax = jnp.max(s, axis=1)
    rmax_ref[0, 0] = rmax
    col_iota = lax.broadcasted_iota(jnp.int32, (m, n), 1)
    cand = jnp.where(s == rmax[:, None], col_iota, jnp.int32(2147483647))
    rarg_ref[0, 0] = jnp.min(cand, axis=1)
    cmax_ref[0, 0] = jnp.max(s, axis=0)


def _pass2_body(rmax_ref, rarg_ref, cmax_ref, conf_ref):
    rm = rmax_ref[0, 0]
    ra = rarg_ref[0, 0]
    cm = cmax_ref[0, 0]
    m = rm.shape[0]
    n = cm.shape[0]
    col_iota = lax.broadcasted_iota(jnp.int32, (m, n), 1)
    mut = jnp.logical_and(col_iota == ra[:, None], rm[:, None] == cm[None, :])
    sig = jnp.where(rm >= _THRESH, jax.nn.sigmoid(rm), 0.0)
    conf_ref[0] = jnp.where(mut, jnp.broadcast_to(sig[:, None], (m, n)), 0.0)


def kernel(desc_A, desc_B, matchability_A, matchability_B):
    B, M, D = desc_A.shape
    N = desc_B.shape[1]
    mA3 = matchability_A.reshape(B, 1, M)
    mB3 = matchability_B.reshape(B, 1, N)

    p1 = pl.pallas_call(
        functools.partial(_pass1_body, precision=lax.Precision.DEFAULT),
        grid=(B,),
        in_specs=[
            pl.BlockSpec((1, M, D), lambda b: (b, 0, 0)),
            pl.BlockSpec((1, N, D), lambda b: (b, 0, 0)),
            pl.BlockSpec((1, 1, M), lambda b: (b, 0, 0)),
            pl.BlockSpec((1, 1, N), lambda b: (b, 0, 0)),
        ],
        out_specs=[
            pl.BlockSpec((1, M, N), lambda b: (b, 0, 0)),
            pl.BlockSpec((1, 1, M), lambda b: (b, 0, 0)),
            pl.BlockSpec((1, 1, M), lambda b: (b, 0, 0)),
            pl.BlockSpec((1, 1, N), lambda b: (b, 0, 0)),
        ],
        out_shape=[
            jax.ShapeDtypeStruct((B, M, N), jnp.float32),
            jax.ShapeDtypeStruct((B, 1, M), jnp.float32),
            jax.ShapeDtypeStruct((B, 1, M), jnp.int32),
            jax.ShapeDtypeStruct((B, 1, N), jnp.float32),
        ],
        compiler_params=pltpu.CompilerParams(
            dimension_semantics=("arbitrary",)),
    )
    scores, rowmax, rowarg, colmax = p1(desc_A, desc_B, mA3, mB3)

    confidence = _sc_confidence(
        rowmax.reshape(B * M), rowarg.reshape(B * M),
        colmax.reshape(B, N), B, M, N)
    return scores, confidence


def _sc_confidence(rmax_flat, rarg_flat, cmax, B, M, N):
    NC, NS = 2, 16
    NW = NC * NS
    rows_w = (B * M) // NW
    zwords = 16384
    nz = (rows_w * N) // zwords
    mesh = plsc.VectorSubcoreMesh(core_axis_name="c", subcore_axis_name="s")

    @functools.partial(
        pl.kernel, mesh=mesh,
        out_type=jax.ShapeDtypeStruct((B * M * N,), jnp.float32),
        compiler_params=pltpu.CompilerParams(needs_layout_passes=False),
        scratch_types=[
            pltpu.VMEM((zwords,), jnp.float32),
            pltpu.VMEM((rows_w,), jnp.float32),
            pltpu.VMEM((rows_w,), jnp.int32),
            pltpu.VMEM((N,), jnp.float32),
            pltpu.VMEM((2, 128), jnp.int32),
            pltpu.VMEM((2, 128), jnp.float32),
            pltpu.SemaphoreType.DMA,
        ],
    )
    def k(rmax_hbm, rarg_hbm, cmax_hbm, out_hbm,
          zbuf, rm_v, ra_v, cm_v, offs_v, vals_v, sem):
        wid = lax.axis_index("s") * NC + lax.axis_index("c")
        row0 = wid * rows_w
        b = row0 // M
        zeros16 = jnp.zeros((16,), jnp.float32)

        def _init(t, carry):
            zbuf[pl.ds(t * 16, 16)] = zeros16
            return carry
        lax.fori_loop(0, zwords // 16, _init, 0)

        pltpu.sync_copy(rmax_hbm.at[pl.ds(row0, rows_w)], rm_v)
        pltpu.sync_copy(rarg_hbm.at[pl.ds(row0, rows_w)], ra_v)
        pltpu.sync_copy(cmax_hbm.at[b], cm_v)

        w0 = row0 * N

        def _zfill(t, carry):
            pltpu.sync_copy(zbuf, out_hbm.at[pl.ds(w0 + t * zwords, zwords)])
            return carry
        lax.fori_loop(0, nz, _zfill, 0)

        lane = lax.iota(jnp.int32, 16)
        for v in range(rows_w // 16):
            rm = rm_v[pl.ds(v * 16, 16)]
            ra = ra_v[pl.ds(v * 16, 16)]
            cg = plsc.load_gather(cm_v, [ra])
            ok = jnp.logical_and(rm == cg, rm >= _THRESH)
            sig = 1.0 / (1.0 + jnp.exp(-rm))
            val = jnp.where(ok, sig, 0.0)
            rows = row0 + v * 16 + lane
            off = rows * N + ra
            j, sl = divmod(v * 16, 128)
            offs_v[j, pl.ds(sl, 16)] = off
            vals_v[j, pl.ds(sl, 16)] = val
        for j in range(2):
            pltpu.async_copy(vals_v.at[j], out_hbm.at[offs_v.at[j]], sem).wait()

    out = k(rmax_flat, rarg_flat, cmax)
    return out.reshape(B, M, N)


# speedup vs baseline: 1.0092x; 1.0092x over previous
import functools

import jax
import jax.numpy as jnp
from jax import lax
from jax.experimental import pallas as pl
from jax.experimental.pallas import tpu as pltpu
from jax.experimental.pallas import tpu_sc as plsc

_THRESH = -3.0


def _pass1_body(dA_ref, dB_ref, mA_ref, mB_ref,
                scores_ref, rmax_ref, rarg_ref, cmax_ref, *, precision):
    dB = dB_ref[0]
    nB = jnp.sqrt(jnp.sum(dB * dB, axis=-1, keepdims=True))
    dBn = dB / (nB + 1e-12)

    dA = dA_ref[0]
    nA = jnp.sqrt(jnp.sum(dA * dA, axis=-1, keepdims=True))
    d = dA.shape[-1]
    dAn = (dA / (nA + 1e-12)) * (float(d) ** 0.5)

    s = lax.dot_general(dAn, dBn, (((1,), (1,)), ((), ())),
                        precision=precision,
                        preferred_element_type=jnp.float32)
    s = (s + mA_ref[0, 0][:, None]) + mB_ref[0, 0][None, :]
    scores_ref[0] = s

    m, n = s.shape
    rmax = jnp.max(s, axis=1)
    rmax_ref[0, 0] = rmax
    col_iota = lax.broadcasted_iota(jnp.int32, (m, n), 1)
    cand = jnp.where(s == rmax[:, None], col_iota, jnp.int32(2147483647))
    rarg_ref[0, 0] = jnp.min(cand, axis=1)
    cmax_ref[0, 0] = jnp.max(s, axis=0)


def _pass2_body(rmax_ref, rarg_ref, cmax_ref, conf_ref):
    rm = rmax_ref[0, 0]
    ra = rarg_ref[0, 0]
    cm = cmax_ref[0, 0]
    m = rm.shape[0]
    n = cm.shape[0]
    col_iota = lax.broadcasted_iota(jnp.int32, (m, n), 1)
    mut = jnp.logical_and(col_iota == ra[:, None], rm[:, None] == cm[None, :])
    sig = jnp.where(rm >= _THRESH, jax.nn.sigmoid(rm), 0.0)
    conf_ref[0] = jnp.where(mut, jnp.broadcast_to(sig[:, None], (m, n)), 0.0)


def kernel(desc_A, desc_B, matchability_A, matchability_B):
    B, M, D = desc_A.shape
    N = desc_B.shape[1]
    mA3 = matchability_A.reshape(B, 1, M)
    mB3 = matchability_B.reshape(B, 1, N)

    p1 = pl.pallas_call(
        functools.partial(_pass1_body, precision=lax.Precision.DEFAULT),
        grid=(B,),
        in_specs=[
            pl.BlockSpec((1, M, D), lambda b: (b, 0, 0)),
            pl.BlockSpec((1, N, D), lambda b: (b, 0, 0)),
            pl.BlockSpec((1, 1, M), lambda b: (b, 0, 0)),
            pl.BlockSpec((1, 1, N), lambda b: (b, 0, 0)),
        ],
        out_specs=[
            pl.BlockSpec((1, M, N), lambda b: (b, 0, 0)),
            pl.BlockSpec((1, 1, M), lambda b: (b, 0, 0)),
            pl.BlockSpec((1, 1, M), lambda b: (b, 0, 0)),
            pl.BlockSpec((1, 1, N), lambda b: (b, 0, 0)),
        ],
        out_shape=[
            jax.ShapeDtypeStruct((B, M, N), jnp.float32),
            jax.ShapeDtypeStruct((B, 1, M), jnp.float32),
            jax.ShapeDtypeStruct((B, 1, M), jnp.int32),
            jax.ShapeDtypeStruct((B, 1, N), jnp.float32),
        ],
        compiler_params=pltpu.CompilerParams(
            dimension_semantics=("arbitrary",)),
    )
    scores, rowmax, rowarg, colmax = p1(desc_A, desc_B, mA3, mB3)

    confidence = _sc_confidence(
        rowmax.reshape(B * M), rowarg.reshape(B * M),
        colmax.reshape(B, N), B, M, N)
    return scores, confidence


def _sc_confidence(rmax_flat, rarg_flat, cmax, B, M, N):
    NC, NS = 2, 16
    NW = NC * NS
    rows_w = (B * M) // NW
    zwords = 16384
    nz = (rows_w * N) // zwords
    mesh = plsc.VectorSubcoreMesh(core_axis_name="c", subcore_axis_name="s")

    @functools.partial(
        pl.kernel, mesh=mesh,
        out_type=jax.ShapeDtypeStruct((B * M * N,), jnp.float32),
        compiler_params=pltpu.CompilerParams(needs_layout_passes=False),
        scratch_types=[
            pltpu.VMEM((zwords,), jnp.float32),
            pltpu.VMEM((rows_w,), jnp.float32),
            pltpu.VMEM((rows_w,), jnp.int32),
            pltpu.VMEM((N,), jnp.float32),
            pltpu.VMEM((2, 128), jnp.int32),
            pltpu.VMEM((2, 128), jnp.float32),
            pltpu.SemaphoreType.DMA,
        ],
    )
    def k(rmax_hbm, rarg_hbm, cmax_hbm, out_hbm,
          zbuf, rm_v, ra_v, cm_v, offs_v, vals_v, sem):
        wid = lax.axis_index("s") * NC + lax.axis_index("c")
        row0 = wid * rows_w
        b = row0 // M
        zeros16 = jnp.zeros((16,), jnp.float32)

        def _init(t, carry):
            zbuf[pl.ds(t * 16, 16)] = zeros16
            return carry
        lax.fori_loop(0, zwords // 16, _init, 0)

        pltpu.sync_copy(rmax_hbm.at[pl.ds(row0, rows_w)], rm_v)
        pltpu.sync_copy(rarg_hbm.at[pl.ds(row0, rows_w)], ra_v)
        pltpu.sync_copy(cmax_hbm.at[b], cm_v)

        w0 = row0 * N
        fills = [
            pltpu.async_copy(
                zbuf, out_hbm.at[pl.ds(w0 + t * zwords, zwords)], sem)
            for t in range(nz)
        ]
        for f in fills:
            f.wait()

        lane = lax.iota(jnp.int32, 16)
        for v in range(rows_w // 16):
            rm = rm_v[pl.ds(v * 16, 16)]
            ra = ra_v[pl.ds(v * 16, 16)]
            cg = plsc.load_gather(cm_v, [ra])
            ok = jnp.logical_and(rm == cg, rm >= _THRESH)
            sig = 1.0 / (1.0 + jnp.exp(-rm))
            val = jnp.where(ok, sig, 0.0)
            rows = row0 + v * 16 + lane
            off = rows * N + ra
            j, sl = divmod(v * 16, 128)
            offs_v[j, pl.ds(sl, 16)] = off
            vals_v[j, pl.ds(sl, 16)] = val
        for j in range(2):
            pltpu.async_copy(vals_v.at[j], out_hbm.at[offs_v.at[j]], sem).wait()

    out = k(rmax_flat, rarg_flat, cmax)
    return out.reshape(B, M, N)


# speedup vs baseline: 1.5593x; 1.5451x over previous
import functools

import jax
import jax.numpy as jnp
from jax import lax
from jax.experimental import pallas as pl
from jax.experimental.pallas import tpu as pltpu

_THRESH = -3.0
_NQ = 4


def _body(dA_ref, dB_ref, mA_ref, mB_ref, scores_ref, conf_ref,
          dAn_scr, rmax_scr, rarg_scr, cmax_scrs, *, bn, precision):
    j = pl.program_id(1)

    @pl.when(j == 0)
    def _():
        dA = dA_ref[0]
        nA = jnp.sqrt(jnp.sum(dA * dA, axis=-1, keepdims=True))
        d = dA.shape[-1]
        dAn_scr[...] = (dA / (nA + 1e-12)) * (float(d) ** 0.5)

    for q in range(_NQ):
        @pl.when(j == q)
        def _(q=q):
            dB = dB_ref[0]
            nB = jnp.sqrt(jnp.sum(dB * dB, axis=-1, keepdims=True))
            dBn = dB / (nB + 1e-12)
            s = lax.dot_general(dAn_scr[...], dBn, (((1,), (1,)), ((), ())),
                                precision=precision,
                                preferred_element_type=jnp.float32)
            s = (s + mA_ref[0, 0][:, None]) + mB_ref[0, 0][None, :]
            scores_ref[0] = s

            m = s.shape[0]
            rmax_q = jnp.max(s, axis=1)
            col_iota = lax.broadcasted_iota(jnp.int32, (m, bn), 1)
            cand = jnp.where(s == rmax_q[:, None], col_iota,
                             jnp.int32(2147483647))
            rarg_q = jnp.min(cand, axis=1) + q * bn
            cmax_scrs[q][...] = jnp.max(s, axis=0)[None, :]

            if q == 0:
                rmax_scr[...] = rmax_q[None, :]
                rarg_scr[...] = rarg_q[None, :]
            else:
                prev_max = rmax_scr[0]
                prev_arg = rarg_scr[0]
                upd = rmax_q > prev_max
                rmax_scr[...] = jnp.where(upd, rmax_q, prev_max)[None, :]
                rarg_scr[...] = jnp.where(upd, rarg_q, prev_arg)[None, :]

        @pl.when(j == q + _NQ)
        def _(q=q):
            rm = rmax_scr[0]
            ra = rarg_scr[0]
            cm = cmax_scrs[q][0]
            m = rm.shape[0]
            ci = lax.broadcasted_iota(jnp.int32, (m, bn), 1) + q * bn
            mut = jnp.logical_and(ci == ra[:, None],
                                  rm[:, None] == cm[None, :])
            sig = jnp.where(rm >= _THRESH, jax.nn.sigmoid(rm), 0.0)
            conf_ref[0] = jnp.where(
                mut, jnp.broadcast_to(sig[:, None], (m, bn)), 0.0)


def kernel(desc_A, desc_B, matchability_A, matchability_B):
    B, M, D = desc_A.shape
    N = desc_B.shape[1]
    bn = N // _NQ
    mA3 = matchability_A.reshape(B, 1, M)
    mB3 = matchability_B.reshape(B, 1, N)

    def _scr(*a):
        pass

    p = pl.pallas_call(
        functools.partial(_body, bn=bn, precision=lax.Precision.DEFAULT),
        grid=(B, 2 * _NQ),
        in_specs=[
            pl.BlockSpec((1, M, D), lambda b, j: (b, 0, 0)),
            pl.BlockSpec((1, bn, D),
                         lambda b, j: (b, jnp.minimum(j, _NQ - 1), 0)),
            pl.BlockSpec((1, 1, M), lambda b, j: (b, 0, 0)),
            pl.BlockSpec((1, 1, bn),
                         lambda b, j: (b, 0, jnp.minimum(j, _NQ - 1))),
        ],
        out_specs=[
            pl.BlockSpec((1, M, bn),
                         lambda b, j: (b, 0, jnp.minimum(j, _NQ - 1))),
            pl.BlockSpec((1, M, bn),
                         lambda b, j: (b, 0, jnp.maximum(j - _NQ, 0))),
        ],
        out_shape=[
            jax.ShapeDtypeStruct((B, M, N), jnp.float32),
            jax.ShapeDtypeStruct((B, M, N), jnp.float32),
        ],
        scratch_shapes=[
            pltpu.VMEM((M, D), jnp.float32),
            pltpu.VMEM((1, M), jnp.float32),
            pltpu.VMEM((1, M), jnp.int32),
            [pltpu.VMEM((1, bn), jnp.float32) for _ in range(_NQ)],
        ],
        compiler_params=pltpu.CompilerParams(
            dimension_semantics=("arbitrary", "arbitrary")),
    )
    scores, confidence = p(desc_A, desc_B, mA3, mB3)
    return scores, confidence


# speedup vs baseline: 2.4215x; 1.5530x over previous
import functools

import jax
import jax.numpy as jnp
from jax import lax
from jax.experimental import pallas as pl
from jax.experimental.pallas import tpu as pltpu

_THRESH = -3.0


def _pass1_body(dA_ref, dB_ref, mA_ref, mB_ref,
                scores_ref, rmax_ref, rarg_ref, cmax_ref, *, precision):
    dB = dB_ref[0]
    nB = jnp.sqrt(jnp.sum(dB * dB, axis=-1, keepdims=True))
    dBn = dB / (nB + 1e-12)

    dA = dA_ref[0]
    nA = jnp.sqrt(jnp.sum(dA * dA, axis=-1, keepdims=True))
    d = dA.shape[-1]
    dAn = (dA / (nA + 1e-12)) * (float(d) ** 0.5)

    s = lax.dot_general(dAn, dBn, (((1,), (1,)), ((), ())),
                        precision=precision,
                        preferred_element_type=jnp.float32)
    s = (s + mA_ref[0, 0][:, None]) + mB_ref[0, 0][None, :]
    scores_ref[0] = s

    m, n = s.shape
    rmax = jnp.max(s, axis=1)
    rmax_ref[0, 0] = rmax
    col_iota = lax.broadcasted_iota(jnp.int32, (m, n), 1)
    cand = jnp.where(s == rmax[:, None], col_iota, jnp.int32(2147483647))
    rarg_ref[0, 0] = jnp.min(cand, axis=1)
    cmax_ref[0, 0] = jnp.max(s, axis=0)


def _pass2_body(rmax_ref, rarg_ref, cmax_ref, conf_ref):
    rm = rmax_ref[0, 0]
    ra = rarg_ref[0, 0]
    cm = cmax_ref[0, 0]
    m = rm.shape[0]
    n = cm.shape[0]
    col_iota = lax.broadcasted_iota(jnp.int32, (m, n), 1)
    mut = jnp.logical_and(col_iota == ra[:, None], rm[:, None] == cm[None, :])
    sig = jnp.where(rm >= _THRESH, jax.nn.sigmoid(rm), 0.0)
    conf_ref[0] = jnp.where(mut, jnp.broadcast_to(sig[:, None], (m, n)), 0.0)


def kernel(desc_A, desc_B, matchability_A, matchability_B):
    B, M, D = desc_A.shape
    N = desc_B.shape[1]
    mA3 = matchability_A.reshape(B, 1, M)
    mB3 = matchability_B.reshape(B, 1, N)

    p1 = pl.pallas_call(
        functools.partial(_pass1_body, precision=lax.Precision.DEFAULT),
        grid=(B,),
        in_specs=[
            pl.BlockSpec((1, M, D), lambda b: (b, 0, 0)),
            pl.BlockSpec((1, N, D), lambda b: (b, 0, 0)),
            pl.BlockSpec((1, 1, M), lambda b: (b, 0, 0)),
            pl.BlockSpec((1, 1, N), lambda b: (b, 0, 0)),
        ],
        out_specs=[
            pl.BlockSpec((1, M, N), lambda b: (b, 0, 0)),
            pl.BlockSpec((1, 1, M), lambda b: (b, 0, 0)),
            pl.BlockSpec((1, 1, M), lambda b: (b, 0, 0)),
            pl.BlockSpec((1, 1, N), lambda b: (b, 0, 0)),
        ],
        out_shape=[
            jax.ShapeDtypeStruct((B, M, N), jnp.float32),
            jax.ShapeDtypeStruct((B, 1, M), jnp.float32),
            jax.ShapeDtypeStruct((B, 1, M), jnp.int32),
            jax.ShapeDtypeStruct((B, 1, N), jnp.float32),
        ],
        compiler_params=pltpu.CompilerParams(
            dimension_semantics=("arbitrary",)),
    )
    scores, rowmax, rowarg, colmax = p1(desc_A, desc_B, mA3, mB3)

    p2 = pl.pallas_call(
        _pass2_body,
        grid=(B,),
        in_specs=[
            pl.BlockSpec((1, 1, M), lambda b: (b, 0, 0)),
            pl.BlockSpec((1, 1, M), lambda b: (b, 0, 0)),
            pl.BlockSpec((1, 1, N), lambda b: (b, 0, 0)),
        ],
        out_specs=pl.BlockSpec((1, M, N), lambda b: (b, 0, 0)),
        out_shape=jax.ShapeDtypeStruct((B, M, N), jnp.float32),
        compiler_params=pltpu.CompilerParams(
            dimension_semantics=("arbitrary",)),
    )
    confidence = p2(rowmax, rowarg, colmax)
    return scores, confidence
